# baseline (speedup 1.0000x reference)
import jax
import jax.numpy as jnp
from jax.experimental import pallas as pl
from jax.experimental.pallas import tpu as pltpu
from jax.experimental.pallas import tpu_sc as plsc

K_TILE = 512
GATHER_W = 128
ROW_SPLIT = 4


def _argmin_tc(xr, keys2, e_sq, k_sq, b, n, k_total):
    k_tiles = k_total // K_TILE
    d = xr.shape[1]
    CH = 8
    n_ch = K_TILE // CH
    SUB = 4
    sub_rows = K_TILE // SUB

    def body(x_ref, keys_ref, esq_ref, ksq_ref, out_ref,
             xbf_ref, kbf_ref, minval_ref, minidx_ref):
        kt = pl.program_id(0)
        bi = pl.program_id(1)

        @pl.when(kt == 0)
        def _cast_x():
            xbf_ref[bi] = x_ref[bi].astype(jnp.bfloat16)

        @pl.when(bi == 0)
        def _cast_keys():
            kbf_ref[...] = keys_ref[...].astype(jnp.bfloat16)

        esq = esq_ref[pl.ds(bi, 1), :]
        xb = xbf_ref[bi]

        val = None
        for t in range(SUB):
            cross_t = jnp.dot(kbf_ref[pl.ds(t * sub_rows, sub_rows), :], xb,
                              preferred_element_type=jnp.float32)
            for jj in range(sub_rows // CH):
                j = t * (sub_rows // CH) + jj
                v = cross_t[jj * CH:(jj + 1) * CH, :]
                ksqj = ksq_ref[pl.ds(j * CH, CH), :]
                dj = (esq - (v + v)) + ksqj
                if j == 0:
                    val = dj
                    chunk = jnp.zeros((CH, n), jnp.int32)
                else:
                    lt = dj < val
                    val = jnp.where(lt, dj, val)
                    chunk = jnp.where(lt, jnp.int32(j), chunk)

        iota_s = jax.lax.broadcasted_iota(jnp.int32, (CH, n), 0)
        idx = chunk * CH + iota_s
        half = CH
        while half > 1:
            half //= 2
            v_lo, v_hi = val[:half, :], val[half:, :]
            i_lo, i_hi = idx[:half, :], idx[half:, :]
            better = (v_hi < v_lo) | ((v_hi == v_lo) & (i_hi < i_lo))
            val = jnp.where(better, v_hi, v_lo)
            idx = jnp.where(better, i_hi, i_lo)

        cand = idx + kt * K_TILE

        @pl.when(kt == 0)
        def _init():
            minval_ref[pl.ds(bi, 1), :] = val
            minidx_ref[pl.ds(bi, 1), :] = cand

        @pl.when(kt > 0)
        def _update():
            old_v = minval_ref[pl.ds(bi, 1), :]
            old_i = minidx_ref[pl.ds(bi, 1), :]
            better = val < old_v
            minval_ref[pl.ds(bi, 1), :] = jnp.where(better, val, old_v)
            minidx_ref[pl.ds(bi, 1), :] = jnp.where(better, cand, old_i)

        @pl.when(kt == k_tiles - 1)
        def _emit():
            out_ref[pl.ds(bi, 1), :] = minidx_ref[pl.ds(bi, 1), :]

    return pl.pallas_call(
        body,
        grid=(k_tiles, b),
        in_specs=[
            pl.BlockSpec((b, d, n), lambda kt, bi: (0, 0, 0)),
            pl.BlockSpec((K_TILE, d), lambda kt, bi: (kt, 0)),
            pl.BlockSpec((b, n), lambda kt, bi: (0, 0)),
            pl.BlockSpec((K_TILE, n), lambda kt, bi: (kt, 0)),
        ],
        out_specs=pl.BlockSpec((b, n), lambda kt, bi: (0, 0)),
        out_shape=jax.ShapeDtypeStruct((b, n), jnp.int32),
        scratch_shapes=[
            pltpu.VMEM((b, d, n), jnp.bfloat16),
            pltpu.VMEM((K_TILE, d), jnp.bfloat16),
            pltpu.VMEM((b, n), jnp.float32),
            pltpu.VMEM((b, n), jnp.int32),
        ],
        compiler_params=pltpu.CompilerParams(
            dimension_semantics=("arbitrary", "arbitrary"),
        ),
    )(xr, keys2, e_sq, k_sq)


def _gather_sc(values2, idx_flat, d):
    n_tot = idx_flat.shape[0]
    sub_d = d // ROW_SPLIT
    n_sub = n_tot * ROW_SPLIT
    mesh = plsc.VectorSubcoreMesh(core_axis_name="core",
                                  subcore_axis_name="subcore")
    vals_sub = values2.reshape(values2.shape[0] * ROW_SPLIT, sub_d)
    idx_sub = (idx_flat[:, None] * ROW_SPLIT
               + jnp.arange(ROW_SPLIT, dtype=jnp.int32)[None, :])
    idx_sub = idx_sub.reshape(1, n_sub)

    @pl.kernel(out_type=jax.ShapeDtypeStruct((n_sub, sub_d), jnp.float32),
               mesh=mesh)
    def gk(values_hbm, i_hbm, o_hbm):
        def gather_body(i_vmem, o_vmem):
            pltpu.sync_copy(values_hbm.at[i_vmem.at[0]], o_vmem)

        pltpu.emit_pipeline(
            gather_body,
            grid=(n_sub // GATHER_W,),
            in_specs=[pl.BlockSpec((1, GATHER_W), index_map=lambda i: (0, i))],
            out_specs=[pl.BlockSpec((GATHER_W, sub_d), index_map=lambda i: (i, 0))],
            core_axis_name=("core", "subcore"),
            dimension_semantics=(pltpu.PARALLEL,),
        )(i_hbm, o_hbm)

    return gk(vals_sub, idx_sub).reshape(n_tot, d)


def kernel(x, keys, values):
    b, d, s, _ = x.shape
    n = s * s
    k_total = keys.shape[1]

    xr = x.reshape(b, d, n)
    keys2 = keys[0]

    emb = jnp.transpose(xr, (0, 2, 1))
    e_sq = jnp.sum(emb * emb, axis=-1)
    k_sq = jnp.sum(keys2 * keys2, axis=-1)
    k_sq = jnp.broadcast_to(k_sq.reshape(k_total, 1), (k_total, n))

    idx = _argmin_tc(xr, keys2, e_sq, k_sq, b, n, k_total)

    mem = _gather_sc(values[0], idx.reshape(b * n), d)

    out = jnp.transpose(mem.reshape(b, n, d), (0, 2, 1)).reshape(b, d, s, s)
    return out


# speedup vs baseline: 1.5181x; 1.5181x over previous
import jax
import jax.numpy as jnp
from jax.experimental import pallas as pl
from jax.experimental.pallas import tpu as pltpu
from jax.experimental.pallas import tpu_sc as plsc

K_TILE = 1024
GATHER_W = 32


def _argmin_tc(xr, keys2, e_sq, k_sq, b, n, k_total):
    k_tiles = k_total // K_TILE
    d = xr.shape[1]
    CH = 8
    n_ch = K_TILE // CH
    SUB = 8
    sub_rows = K_TILE // SUB

    def body(x_ref, keys_ref, esq_ref, ksq_ref, out_ref,
             xbf_ref, kbf_ref, minval_ref, minidx_ref):
        kt = pl.program_id(0)
        bi = pl.program_id(1)

        @pl.when(kt == 0)
        def _cast_x():
            xbf_ref[bi] = x_ref[bi].astype(jnp.bfloat16)

        @pl.when(bi == 0)
        def _cast_keys():
            kbf_ref[...] = keys_ref[...].astype(jnp.bfloat16)

        esq = esq_ref[pl.ds(bi, 1), :]
        xb = xbf_ref[bi]

        acc = []
        for t in range(SUB):
            cross_t = jnp.dot(kbf_ref[pl.ds(t * sub_rows, sub_rows), :], xb,
                              preferred_element_type=jnp.float32)
            val_t = None
            for jj in range(sub_rows // CH):
                j = t * (sub_rows // CH) + jj
                v = cross_t[jj * CH:(jj + 1) * CH, :]
                ksqj = ksq_ref[pl.ds(j * CH, CH), :]
                dj = (esq - (v + v)) + ksqj
                if jj == 0:
                    val_t = dj
                    chunk_t = jnp.full((CH, n), j, jnp.int32)
                else:
                    lt = dj < val_t
                    val_t = jnp.where(lt, dj, val_t)
                    chunk_t = jnp.where(lt, jnp.int32(j), chunk_t)
            acc.append((val_t, chunk_t))

        val, chunk = acc[0]
        for val_t, chunk_t in acc[1:]:
            lt = val_t < val
            val = jnp.where(lt, val_t, val)
            chunk = jnp.where(lt, chunk_t, chunk)

        iota_s = jax.lax.broadcasted_iota(jnp.int32, (CH, n), 0)
        idx = chunk * CH + iota_s
        half = CH
        while half > 1:
            half //= 2
            v_lo, v_hi = val[:half, :], val[half:, :]
            i_lo, i_hi = idx[:half, :], idx[half:, :]
            better = (v_hi < v_lo) | ((v_hi == v_lo) & (i_hi < i_lo))
            val = jnp.where(better, v_hi, v_lo)
            idx = jnp.where(better, i_hi, i_lo)

        cand = idx + kt * K_TILE

        @pl.when(kt == 0)
        def _init():
            minval_ref[pl.ds(bi, 1), :] = val
            minidx_ref[pl.ds(bi, 1), :] = cand

        @pl.when(kt > 0)
        def _update():
            old_v = minval_ref[pl.ds(bi, 1), :]
            old_i = minidx_ref[pl.ds(bi, 1), :]
            better = val < old_v
            minval_ref[pl.ds(bi, 1), :] = jnp.where(better, val, old_v)
            minidx_ref[pl.ds(bi, 1), :] = jnp.where(better, cand, old_i)

        @pl.when(kt == k_tiles - 1)
        def _emit():
            out_ref[pl.ds(bi, 1), :] = minidx_ref[pl.ds(bi, 1), :]

    return pl.pallas_call(
        body,
        grid=(k_tiles, b),
        in_specs=[
            pl.BlockSpec((b, d, n), lambda kt, bi: (0, 0, 0)),
            pl.BlockSpec((K_TILE, d), lambda kt, bi: (kt, 0)),
            pl.BlockSpec((b, n), lambda kt, bi: (0, 0)),
            pl.BlockSpec((K_TILE, n), lambda kt, bi: (kt, 0)),
        ],
        out_specs=pl.BlockSpec((b, n), lambda kt, bi: (0, 0)),
        out_shape=jax.ShapeDtypeStruct((b, n), jnp.int32),
        scratch_shapes=[
            pltpu.VMEM((b, d, n), jnp.bfloat16),
            pltpu.VMEM((K_TILE, d), jnp.bfloat16),
            pltpu.VMEM((b, n), jnp.float32),
            pltpu.VMEM((b, n), jnp.int32),
        ],
        compiler_params=pltpu.CompilerParams(
            dimension_semantics=("arbitrary", "arbitrary"),
        ),
    )(xr, keys2, e_sq, k_sq)


def _gather_sc(values2, idx_flat, d):
    n_tot = idx_flat.shape[0]
    n_win = n_tot // GATHER_W
    mesh = plsc.VectorSubcoreMesh(core_axis_name="core",
                                  subcore_axis_name="subcore")
    idx_rows = jnp.zeros((n_win, 128), jnp.int32)
    idx_rows = idx_rows.at[:, :GATHER_W].set(idx_flat.reshape(n_win, GATHER_W))

    @pl.kernel(out_type=jax.ShapeDtypeStruct((n_tot, d), jnp.float32),
               mesh=mesh)
    def gk(values_hbm, i_hbm, o_hbm):
        def gather_body(i_vmem, o_vmem):
            pltpu.sync_copy(values_hbm.at[i_vmem.at[0, pl.ds(0, GATHER_W)]],
                            o_vmem)

        pltpu.emit_pipeline(
            gather_body,
            grid=(n_win,),
            in_specs=[pl.BlockSpec((1, 128), index_map=lambda i: (i, 0))],
            out_specs=[pl.BlockSpec((GATHER_W, d), index_map=lambda i: (i, 0))],
            core_axis_name=("core", "subcore"),
            dimension_semantics=(pltpu.PARALLEL,),
        )(i_hbm, o_hbm)

    return gk(values2, idx_rows)


def kernel(x, keys, values):
    b, d, s, _ = x.shape
    n = s * s
    k_total = keys.shape[1]

    xr = x.reshape(b, d, n)
    keys2 = keys[0]

    emb = jnp.transpose(xr, (0, 2, 1))
    e_sq = jnp.sum(emb * emb, axis=-1)
    k_sq = jnp.sum(keys2 * keys2, axis=-1)
    k_sq = jnp.broadcast_to(k_sq.reshape(k_total, 1), (k_total, n))

    idx = _argmin_tc(xr, keys2, e_sq, k_sq, b, n, k_total)

    mem = _gather_sc(values[0], idx.reshape(b * n), d)

    out = jnp.transpose(mem.reshape(b, n, d), (0, 2, 1)).reshape(b, d, s, s)
    return out


# speedup vs baseline: 1.5765x; 1.0385x over previous
import jax
import jax.numpy as jnp
from jax.experimental import pallas as pl
from jax.experimental.pallas import tpu as pltpu
from jax.experimental.pallas import tpu_sc as plsc

K_TILE = 1024
GATHER_W = 32
CH = 8
CHAINS = 4
LH = 128


def _argmin_tc(xr, keys2, e_sq, k_sq, b, n, k_total):
    k_tiles = k_total // K_TILE
    d = xr.shape[1]
    n_ch = K_TILE // CH
    n_h = n // LH

    def body(x_ref, keys_ref, esq_ref, ksq_ref, out_ref,
             xbf_ref, kbf_ref, minval_ref, minidx_ref):
        kt = pl.program_id(0)
        bi = pl.program_id(1)

        @pl.when(kt == 0)
        def _cast_x():
            xbf_ref[bi] = x_ref[bi].astype(jnp.bfloat16)

        @pl.when(bi == 0)
        def _cast_keys():
            kbf_ref[...] = keys_ref[...].astype(jnp.bfloat16) * jnp.bfloat16(-2)

        xb = xbf_ref[bi]
        cross = jnp.dot(kbf_ref[...], xb,
                        preferred_element_type=jnp.float32)
        esq_row = esq_ref[pl.ds(bi, 1), :]

        half_res = []
        for h in range(n_h):
            esq_h = jnp.broadcast_to(
                esq_row[:, h * LH:(h + 1) * LH], (CH, LH))
            acc = [None] * CHAINS
            for j in range(n_ch):
                v = cross[j * CH:(j + 1) * CH, h * LH:(h + 1) * LH]
                ksqj = ksq_ref[pl.ds(j * CH, CH), :]
                dj = (esq_h + v) + ksqj
                a = j % CHAINS
                if acc[a] is None:
                    acc[a] = (dj, jnp.full((CH, LH), j, jnp.int32))
                else:
                    val_a, chunk_a = acc[a]
                    lt = dj < val_a
                    acc[a] = (jnp.where(lt, dj, val_a),
                              jnp.where(lt, jnp.int32(j), chunk_a))

            val, chunk = acc[0]
            for val_t, chunk_t in acc[1:]:
                better = (val_t < val) | ((val_t == val) & (chunk_t < chunk))
                val = jnp.where(better, val_t, val)
                chunk = jnp.where(better, chunk_t, chunk)

            iota_s = jax.lax.broadcasted_iota(jnp.int32, (CH, LH), 0)
            idx = chunk * CH + iota_s
            half = CH
            while half > 1:
                half //= 2
                v_lo, v_hi = val[:half, :], val[half:, :]
                i_lo, i_hi = idx[:half, :], idx[half:, :]
                better = (v_hi < v_lo) | ((v_hi == v_lo) & (i_hi < i_lo))
                val = jnp.where(better, v_hi, v_lo)
                idx = jnp.where(better, i_hi, i_lo)
            half_res.append((val, idx))

        val = jnp.concatenate([vr for vr, _ in half_res], axis=1)
        idx = jnp.concatenate([ir for _, ir in half_res], axis=1)
        cand = idx + kt * K_TILE

        @pl.when(kt == 0)
        def _init():
            minval_ref[pl.ds(bi, 1), :] = val
            minidx_ref[pl.ds(bi, 1), :] = cand

        @pl.when(kt > 0)
        def _update():
            old_v = minval_ref[pl.ds(bi, 1), :]
            old_i = minidx_ref[pl.ds(bi, 1), :]
            better = val < old_v
            minval_ref[pl.ds(bi, 1), :] = jnp.where(better, val, old_v)
            minidx_ref[pl.ds(bi, 1), :] = jnp.where(better, cand, old_i)

        @pl.when(kt == k_tiles - 1)
        def _emit():
            out_ref[pl.ds(bi, 1), :] = minidx_ref[pl.ds(bi, 1), :]

    return pl.pallas_call(
        body,
        grid=(k_tiles, b),
        in_specs=[
            pl.BlockSpec((b, d, n), lambda kt, bi: (0, 0, 0)),
            pl.BlockSpec((K_TILE, d), lambda kt, bi: (kt, 0)),
            pl.BlockSpec((b, n), lambda kt, bi: (0, 0)),
            pl.BlockSpec((K_TILE, LH), lambda kt, bi: (kt, 0)),
        ],
        out_specs=pl.BlockSpec((b, n), lambda kt, bi: (0, 0)),
        out_shape=jax.ShapeDtypeStruct((b, n), jnp.int32),
        scratch_shapes=[
            pltpu.VMEM((b, d, n), jnp.bfloat16),
            pltpu.VMEM((K_TILE, d), jnp.bfloat16),
            pltpu.VMEM((b, n), jnp.float32),
            pltpu.VMEM((b, n), jnp.int32),
        ],
        compiler_params=pltpu.CompilerParams(
            dimension_semantics=("arbitrary", "arbitrary"),
        ),
    )(xr, keys2, e_sq, k_sq)


def _gather_sc(values3, idx_flat, d):
    n_tot = idx_flat.shape[0]
    n_win = n_tot // GATHER_W
    mesh = plsc.VectorSubcoreMesh(core_axis_name="core",
                                  subcore_axis_name="subcore")
    idx_rows = jnp.zeros((n_win, 128), jnp.int32)
    idx_rows = idx_rows.at[:, :GATHER_W].set(idx_flat.reshape(n_win, GATHER_W))

    @pl.kernel(out_type=jax.ShapeDtypeStruct((n_tot, d), jnp.float32),
               mesh=mesh)
    def gk(values_hbm, i_hbm, o_hbm):
        values2d = values_hbm.at[0]

        def gather_body(i_vmem, o_vmem):
            pltpu.sync_copy(values2d.at[i_vmem.at[0, pl.ds(0, GATHER_W)]],
                            o_vmem)

        pltpu.emit_pipeline(
            gather_body,
            grid=(n_win,),
            in_specs=[pl.BlockSpec((1, 128), index_map=lambda i: (i, 0))],
            out_specs=[pl.BlockSpec((GATHER_W, d), index_map=lambda i: (i, 0))],
            core_axis_name=("core", "subcore"),
            dimension_semantics=(pltpu.PARALLEL,),
        )(i_hbm, o_hbm)

    return gk(values3, idx_rows)


def kernel(x, keys, values):
    b, d, s, _ = x.shape
    n = s * s
    k_total = keys.shape[1]

    xr = x.reshape(b, d, n)
    keys2 = keys[0]

    emb = jnp.transpose(xr, (0, 2, 1))
    e_sq = jnp.sum(emb * emb, axis=-1)
    k_sq = jnp.sum(keys2 * keys2, axis=-1)
    k_sq = jnp.broadcast_to(k_sq.reshape(k_total, 1), (k_total, LH))

    idx = _argmin_tc(xr, keys2, e_sq, k_sq, b, n, k_total)

    mem = _gather_sc(values, idx.reshape(b * n), d)

    out = jnp.transpose(mem.reshape(b, n, d), (0, 2, 1)).reshape(b, d, s, s)
    return out


# speedup vs baseline: 1.6509x; 1.0472x over previous
import jax
import jax.numpy as jnp
from jax.experimental import pallas as pl
from jax.experimental.pallas import tpu as pltpu
from jax.experimental.pallas import tpu_sc as plsc

K_TILE = 1024
GATHER_W = 32
CH = 8
CHAINS = 4
LH = 128
SUB = 2


def _argmin_tc(xbf, kbf, e_sq, k_sq, b, n, k_total):
    k_tiles = k_total // K_TILE
    d = xbf.shape[1]
    n_h = n // LH

    def body(x_ref, keys_ref, esq_ref, ksq_ref, out_ref,
             minval_ref, minidx_ref):
        kt = pl.program_id(0)
        bi = pl.program_id(1)

        xb = x_ref[bi]
        esq_row = esq_ref[pl.ds(bi, 1), :]
        sub_rows = K_TILE // SUB
        sub_ch = sub_rows // CH

        acc = [[None] * CHAINS for _ in range(n_h)]
        for t in range(SUB):
            cross_t = jnp.dot(keys_ref[pl.ds(t * sub_rows, sub_rows), :], xb,
                              preferred_element_type=jnp.float32)
            for h in range(n_h):
                esq_h = jnp.broadcast_to(
                    esq_row[:, h * LH:(h + 1) * LH], (CH, LH))
                for jj in range(sub_ch):
                    j = t * sub_ch + jj
                    v = cross_t[jj * CH:(jj + 1) * CH, h * LH:(h + 1) * LH]
                    ksqj = ksq_ref[pl.ds(j * CH, CH), :]
                    dj = (esq_h + v) + ksqj
                    a = j % CHAINS
                    if acc[h][a] is None:
                        acc[h][a] = (dj, jnp.full((CH, LH), j, jnp.int32))
                    else:
                        val_a, chunk_a = acc[h][a]
                        lt = dj < val_a
                        acc[h][a] = (jnp.where(lt, dj, val_a),
                                     jnp.where(lt, jnp.int32(j), chunk_a))

        half_res = []
        for h in range(n_h):
            val, chunk = acc[h][0]
            for val_t, chunk_t in acc[h][1:]:
                better = (val_t < val) | ((val_t == val) & (chunk_t < chunk))
                val = jnp.where(better, val_t, val)
                chunk = jnp.where(better, chunk_t, chunk)

            iota_s = jax.lax.broadcasted_iota(jnp.int32, (CH, LH), 0)
            idx = chunk * CH + iota_s
            half = CH
            while half > 1:
                half //= 2
                v_lo, v_hi = val[:half, :], val[half:, :]
                i_lo, i_hi = idx[:half, :], idx[half:, :]
                better = (v_hi < v_lo) | ((v_hi == v_lo) & (i_hi < i_lo))
                val = jnp.where(better, v_hi, v_lo)
                idx = jnp.where(better, i_hi, i_lo)
            half_res.append((val, idx))

        val = jnp.concatenate([vr for vr, _ in half_res], axis=1)
        idx = jnp.concatenate([ir for _, ir in half_res], axis=1)
        cand = idx + kt * K_TILE

        @pl.when(kt == 0)
        def _init():
            minval_ref[pl.ds(bi, 1), :] = val
            minidx_ref[pl.ds(bi, 1), :] = cand

        @pl.when(kt > 0)
        def _update():
            old_v = minval_ref[pl.ds(bi, 1), :]
            old_i = minidx_ref[pl.ds(bi, 1), :]
            better = val < old_v
            minval_ref[pl.ds(bi, 1), :] = jnp.where(better, val, old_v)
            minidx_ref[pl.ds(bi, 1), :] = jnp.where(better, cand, old_i)

        @pl.when(kt == k_tiles - 1)
        def _emit():
            out_ref[pl.ds(bi, 1), :] = minidx_ref[pl.ds(bi, 1), :]

    return pl.pallas_call(
        body,
        grid=(k_tiles, b),
        in_specs=[
            pl.BlockSpec((b, d, n), lambda kt, bi: (0, 0, 0)),
            pl.BlockSpec((K_TILE, d), lambda kt, bi: (kt, 0)),
            pl.BlockSpec((b, n), lambda kt, bi: (0, 0)),
            pl.BlockSpec((K_TILE, LH), lambda kt, bi: (kt, 0)),
        ],
        out_specs=pl.BlockSpec((b, n), lambda kt, bi: (0, 0)),
        out_shape=jax.ShapeDtypeStruct((b, n), jnp.int32),
        scratch_shapes=[
            pltpu.VMEM((b, n), jnp.float32),
            pltpu.VMEM((b, n), jnp.int32),
        ],
        compiler_params=pltpu.CompilerParams(
            dimension_semantics=("arbitrary", "arbitrary"),
        ),
    )(xbf, kbf, e_sq, k_sq)


def _gather_sc(values3, idx_flat, d):
    n_tot = idx_flat.shape[0]
    n_win = n_tot // GATHER_W
    mesh = plsc.VectorSubcoreMesh(core_axis_name="core",
                                  subcore_axis_name="subcore")
    idx_rows = jnp.zeros((n_win, 128), jnp.int32)
    idx_rows = idx_rows.at[:, :GATHER_W].set(idx_flat.reshape(n_win, GATHER_W))

    @pl.kernel(out_type=jax.ShapeDtypeStruct((n_tot, d), jnp.float32),
               mesh=mesh)
    def gk(values_hbm, i_hbm, o_hbm):
        values2d = values_hbm.at[0]

        def gather_body(i_vmem, o_vmem):
            pltpu.sync_copy(values2d.at[i_vmem.at[0, pl.ds(0, GATHER_W)]],
                            o_vmem)

        pltpu.emit_pipeline(
            gather_body,
            grid=(n_win,),
            in_specs=[pl.BlockSpec((1, 128), index_map=lambda i: (i, 0))],
            out_specs=[pl.BlockSpec((GATHER_W, d), index_map=lambda i: (i, 0))],
            core_axis_name=("core", "subcore"),
            dimension_semantics=(pltpu.PARALLEL,),
        )(i_hbm, o_hbm)

    return gk(values3, idx_rows)


def kernel(x, keys, values):
    b, d, s, _ = x.shape
    n = s * s
    k_total = keys.shape[1]

    xr = x.reshape(b, d, n)
    keys2 = keys[0]

    emb = jnp.transpose(xr, (0, 2, 1))
    e_sq = jnp.sum(emb * emb, axis=-1)
    k_sq = jnp.sum(keys2 * keys2, axis=-1)
    k_sq = jnp.broadcast_to(k_sq.reshape(k_total, 1), (k_total, LH))

    xbf = xr.astype(jnp.bfloat16)
    kbf = (keys2 * jnp.float32(-2)).astype(jnp.bfloat16)

    idx = _argmin_tc(xbf, kbf, e_sq, k_sq, b, n, k_total)

    mem = _gather_sc(values, idx.reshape(b * n), d)

    out = jnp.transpose(mem.reshape(b, n, d), (0, 2, 1)).reshape(b, d, s, s)
    return out


# speedup vs baseline: 1.8049x; 1.0933x over previous
import jax
import jax.numpy as jnp
from jax.experimental import pallas as pl
from jax.experimental.pallas import tpu as pltpu
from jax.experimental.pallas import tpu_sc as plsc

K_TILE = 2048
GATHER_W = 32
CH = 8
CHAINS = 4
LH = 128
SUB = 4


def _argmin_tc(xbf, kbf, e_sq, k_sq, b, n, k_total):
    k_tiles = k_total // K_TILE
    d = xbf.shape[1]
    n_h = n // LH

    def body(x_ref, keys_ref, esq_ref, ksq_ref, out_ref,
             minval_ref, minidx_ref):
        kt = pl.program_id(0)
        bi = pl.program_id(1)

        xb = x_ref[bi]
        esq_row = esq_ref[pl.ds(bi, 1), :]
        sub_rows = K_TILE // SUB
        sub_ch = sub_rows // CH

        acc = [[None] * CHAINS for _ in range(n_h)]
        for t in range(SUB):
            cross_t = jnp.dot(keys_ref[pl.ds(t * sub_rows, sub_rows), :], xb,
                              preferred_element_type=jnp.float32)
            for h in range(n_h):
                esq_h = jnp.broadcast_to(
                    esq_row[:, h * LH:(h + 1) * LH], (CH, LH))
                for jj in range(sub_ch):
                    j = t * sub_ch + jj
                    v = cross_t[jj * CH:(jj + 1) * CH, h * LH:(h + 1) * LH]
                    ksqj = ksq_ref[pl.ds(j * CH, CH), :]
                    dj = (esq_h + v) + ksqj
                    a = j % CHAINS
                    if acc[h][a] is None:
                        acc[h][a] = (dj, jnp.full((CH, LH), j, jnp.int32))
                    else:
                        val_a, chunk_a = acc[h][a]
                        lt = dj < val_a
                        acc[h][a] = (jnp.where(lt, dj, val_a),
                                     jnp.where(lt, jnp.int32(j), chunk_a))

        half_res = []
        for h in range(n_h):
            val, chunk = acc[h][0]
            for val_t, chunk_t in acc[h][1:]:
                better = (val_t < val) | ((val_t == val) & (chunk_t < chunk))
                val = jnp.where(better, val_t, val)
                chunk = jnp.where(better, chunk_t, chunk)

            iota_s = jax.lax.broadcasted_iota(jnp.int32, (CH, LH), 0)
            idx = chunk * CH + iota_s
            half = CH
            while half > 1:
                half //= 2
                v_lo, v_hi = val[:half, :], val[half:, :]
                i_lo, i_hi = idx[:half, :], idx[half:, :]
                better = (v_hi < v_lo) | ((v_hi == v_lo) & (i_hi < i_lo))
                val = jnp.where(better, v_hi, v_lo)
                idx = jnp.where(better, i_hi, i_lo)
            half_res.append((val, idx))

        val = jnp.concatenate([vr for vr, _ in half_res], axis=1)
        idx = jnp.concatenate([ir for _, ir in half_res], axis=1)
        cand = idx + kt * K_TILE

        @pl.when(kt == 0)
        def _init():
            minval_ref[pl.ds(bi, 1), :] = val
            minidx_ref[pl.ds(bi, 1), :] = cand

        @pl.when(kt > 0)
        def _update():
            old_v = minval_ref[pl.ds(bi, 1), :]
            old_i = minidx_ref[pl.ds(bi, 1), :]
            better = val < old_v
            minval_ref[pl.ds(bi, 1), :] = jnp.where(better, val, old_v)
            minidx_ref[pl.ds(bi, 1), :] = jnp.where(better, cand, old_i)

        @pl.when(kt == k_tiles - 1)
        def _emit():
            out_ref[pl.ds(bi, 1), :] = minidx_ref[pl.ds(bi, 1), :]

    return pl.pallas_call(
        body,
        grid=(k_tiles, b),
        in_specs=[
            pl.BlockSpec((b, d, n), lambda kt, bi: (0, 0, 0)),
            pl.BlockSpec((K_TILE, d), lambda kt, bi: (kt, 0)),
            pl.BlockSpec((b, n), lambda kt, bi: (0, 0)),
            pl.BlockSpec((K_TILE, LH), lambda kt, bi: (kt, 0)),
        ],
        out_specs=pl.BlockSpec((b, n), lambda kt, bi: (0, 0)),
        out_shape=jax.ShapeDtypeStruct((b, n), jnp.int32),
        scratch_shapes=[
            pltpu.VMEM((b, n), jnp.float32),
            pltpu.VMEM((b, n), jnp.int32),
        ],
        compiler_params=pltpu.CompilerParams(
            dimension_semantics=("arbitrary", "arbitrary"),
        ),
    )(xbf, kbf, e_sq, k_sq)


def _gather_sc(values3, idx_flat, d):
    n_tot = idx_flat.shape[0]
    n_win = n_tot // GATHER_W
    mesh = plsc.VectorSubcoreMesh(core_axis_name="core",
                                  subcore_axis_name="subcore")
    idx_rows = jnp.zeros((n_win, 128), jnp.int32)
    idx_rows = idx_rows.at[:, :GATHER_W].set(idx_flat.reshape(n_win, GATHER_W))

    @pl.kernel(out_type=jax.ShapeDtypeStruct((n_tot, d), jnp.float32),
               mesh=mesh)
    def gk(values_hbm, i_hbm, o_hbm):
        values2d = values_hbm.at[0]

        def gather_body(i_vmem, o_vmem):
            pltpu.sync_copy(values2d.at[i_vmem.at[0, pl.ds(0, GATHER_W)]],
                            o_vmem)

        pltpu.emit_pipeline(
            gather_body,
            grid=(n_win,),
            in_specs=[pl.BlockSpec((1, 128), index_map=lambda i: (i, 0))],
            out_specs=[pl.BlockSpec((GATHER_W, d), index_map=lambda i: (i, 0))],
            core_axis_name=("core", "subcore"),
            dimension_semantics=(pltpu.PARALLEL,),
        )(i_hbm, o_hbm)

    return gk(values3, idx_rows)


def kernel(x, keys, values):
    b, d, s, _ = x.shape
    n = s * s
    k_total = keys.shape[1]

    xr = x.reshape(b, d, n)
    keys2 = keys[0]

    emb = jnp.transpose(xr, (0, 2, 1))
    e_sq = jnp.sum(emb * emb, axis=-1)
    k_sq = jnp.sum(keys2 * keys2, axis=-1)
    k_sq = jnp.broadcast_to(k_sq.reshape(k_total, 1), (k_total, LH))

    xbf = xr.astype(jnp.bfloat16)
    kbf = (keys2 * jnp.float32(-2)).astype(jnp.bfloat16)

    idx = _argmin_tc(xbf, kbf, e_sq, k_sq, b, n, k_total)

    mem = _gather_sc(values, idx.reshape(b * n), d)

    out = jnp.transpose(mem.reshape(b, n, d), (0, 2, 1)).reshape(b, d, s, s)
    return out


# speedup vs baseline: 1.9331x; 1.0710x over previous
import jax
import jax.numpy as jnp
from jax.experimental import pallas as pl
from jax.experimental.pallas import tpu as pltpu
from jax.experimental.pallas import tpu_sc as plsc

K_TILE = 4096
GATHER_W = 32
CH = 8
CHAINS = 4
LH = 128
SUB = 4


def _argmin_tc(xbf, kbf, e_sq, k_sq, b, n, k_total):
    k_tiles = k_total // K_TILE
    d = xbf.shape[1]
    n_h = n // LH

    def body(x_ref, keys_ref, esq_ref, ksq_ref, out_ref,
             minval_ref, minidx_ref):
        kt = pl.program_id(0)
        bi = pl.program_id(1)

        xb = x_ref[bi]
        esq_row = esq_ref[pl.ds(bi, 1), :]
        sub_rows = K_TILE // SUB
        sub_ch = sub_rows // CH

        acc = [[None] * CHAINS for _ in range(n_h)]
        for t in range(SUB):
            cross_t = jnp.dot(keys_ref[pl.ds(t * sub_rows, sub_rows), :], xb,
                              preferred_element_type=jnp.float32)
            for h in range(n_h):
                esq_h = jnp.broadcast_to(
                    esq_row[:, h * LH:(h + 1) * LH], (CH, LH))
                for jj in range(sub_ch):
                    j = t * sub_ch + jj
                    v = cross_t[jj * CH:(jj + 1) * CH, h * LH:(h + 1) * LH]
                    ksqj = ksq_ref[pl.ds(j * CH, CH), :]
                    dj = (esq_h + v) + ksqj
                    a = j % CHAINS
                    if acc[h][a] is None:
                        acc[h][a] = (dj, jnp.full((CH, LH), j, jnp.int32))
                    else:
                        val_a, chunk_a = acc[h][a]
                        lt = dj < val_a
                        acc[h][a] = (jnp.where(lt, dj, val_a),
                                     jnp.where(lt, jnp.int32(j), chunk_a))

        half_res = []
        for h in range(n_h):
            val, chunk = acc[h][0]
            for val_t, chunk_t in acc[h][1:]:
                better = (val_t < val) | ((val_t == val) & (chunk_t < chunk))
                val = jnp.where(better, val_t, val)
                chunk = jnp.where(better, chunk_t, chunk)

            iota_s = jax.lax.broadcasted_iota(jnp.int32, (CH, LH), 0)
            idx = chunk * CH + iota_s
            half = CH
            while half > 1:
                half //= 2
                v_lo, v_hi = val[:half, :], val[half:, :]
                i_lo, i_hi = idx[:half, :], idx[half:, :]
                better = (v_hi < v_lo) | ((v_hi == v_lo) & (i_hi < i_lo))
                val = jnp.where(better, v_hi, v_lo)
                idx = jnp.where(better, i_hi, i_lo)
            half_res.append((val, idx))

        val = jnp.concatenate([vr for vr, _ in half_res], axis=1)
        idx = jnp.concatenate([ir for _, ir in half_res], axis=1)
        cand = idx + kt * K_TILE

        @pl.when(kt == 0)
        def _init():
            minval_ref[pl.ds(bi, 1), :] = val
            minidx_ref[pl.ds(bi, 1), :] = cand

        @pl.when(kt > 0)
        def _update():
            old_v = minval_ref[pl.ds(bi, 1), :]
            old_i = minidx_ref[pl.ds(bi, 1), :]
            better = val < old_v
            minval_ref[pl.ds(bi, 1), :] = jnp.where(better, val, old_v)
            minidx_ref[pl.ds(bi, 1), :] = jnp.where(better, cand, old_i)

        @pl.when(kt == k_tiles - 1)
        def _emit():
            out_ref[pl.ds(bi, 1), :] = minidx_ref[pl.ds(bi, 1), :]

    return pl.pallas_call(
        body,
        grid=(k_tiles, b),
        in_specs=[
            pl.BlockSpec((b, d, n), lambda kt, bi: (0, 0, 0)),
            pl.BlockSpec((K_TILE, d), lambda kt, bi: (kt, 0)),
            pl.BlockSpec((b, n), lambda kt, bi: (0, 0)),
            pl.BlockSpec((K_TILE, LH), lambda kt, bi: (kt, 0)),
        ],
        out_specs=pl.BlockSpec((b, n), lambda kt, bi: (0, 0)),
        out_shape=jax.ShapeDtypeStruct((b, n), jnp.int32),
        scratch_shapes=[
            pltpu.VMEM((b, n), jnp.float32),
            pltpu.VMEM((b, n), jnp.int32),
        ],
        compiler_params=pltpu.CompilerParams(
            dimension_semantics=("arbitrary", "arbitrary"),
        ),
    )(xbf, kbf, e_sq, k_sq)


def _gather_sc(values3, idx_flat, d):
    n_tot = idx_flat.shape[0]
    n_win = n_tot // GATHER_W
    mesh = plsc.VectorSubcoreMesh(core_axis_name="core",
                                  subcore_axis_name="subcore")
    idx_rows = jnp.zeros((n_win, 128), jnp.int32)
    idx_rows = idx_rows.at[:, :GATHER_W].set(idx_flat.reshape(n_win, GATHER_W))

    @pl.kernel(out_type=jax.ShapeDtypeStruct((n_tot, d), jnp.float32),
               mesh=mesh)
    def gk(values_hbm, i_hbm, o_hbm):
        values2d = values_hbm.at[0]

        def gather_body(i_vmem, o_vmem):
            pltpu.sync_copy(values2d.at[i_vmem.at[0, pl.ds(0, GATHER_W)]],
                            o_vmem)

        pltpu.emit_pipeline(
            gather_body,
            grid=(n_win,),
            in_specs=[pl.BlockSpec((1, 128), index_map=lambda i: (i, 0))],
            out_specs=[pl.BlockSpec((GATHER_W, d), index_map=lambda i: (i, 0))],
            core_axis_name=("core", "subcore"),
            dimension_semantics=(pltpu.PARALLEL,),
        )(i_hbm, o_hbm)

    return gk(values3, idx_rows)


def kernel(x, keys, values):
    b, d, s, _ = x.shape
    n = s * s
    k_total = keys.shape[1]

    xr = x.reshape(b, d, n)
    keys2 = keys[0]

    emb = jnp.transpose(xr, (0, 2, 1))
    e_sq = jnp.sum(emb * emb, axis=-1)
    k_sq = jnp.sum(keys2 * keys2, axis=-1)
    k_sq = jnp.broadcast_to(k_sq.reshape(k_total, 1), (k_total, LH))

    xbf = xr.astype(jnp.bfloat16)
    kbf = (keys2 * jnp.float32(-2)).astype(jnp.bfloat16)

    idx = _argmin_tc(xbf, kbf, e_sq, k_sq, b, n, k_total)

    mem = _gather_sc(values, idx.reshape(b * n), d)

    out = jnp.transpose(mem.reshape(b, n, d), (0, 2, 1)).reshape(b, d, s, s)
    return out
